# baseline (speedup 1.0000x reference)
import jax
import jax.numpy as jnp
from jax import lax
from jax.experimental import pallas as pl
from jax.experimental.pallas import tpu as pltpu
from jax.experimental.pallas import tpu_sc as plsc

_B, _L, _D = 1024, 2048, 128
_N = _B * _L
_NQ = _N // 4
_NC, _NS = 2, 16
_NW = _NC * _NS
_QPW = _NQ // _NW
_QC = 64
_QIDXBLK = 2048
_NBLK = _QPW // _QIDXBLK
_PAIRS = _QIDXBLK // (2 * _QC)


def _emb_body(i0_hbm, i1_hbm, i2_hbm, i3_hbm, qtable_hbm, out_hbm,
              s0, s1, s2, s3, qidx_v, rows0, rows1,
              gsem0, gsem1, wsem0, wsem1):
    wid = lax.axis_index("s") * _NC + lax.axis_index("c")
    qbase = wid * _QPW
    rows = (rows0, rows1)
    gsem = (gsem0, gsem1)
    wsem = (wsem0, wsem1)

    def blk(ob, carry):
        blk_off = qbase + ob * _QIDXBLK
        pltpu.sync_copy(i0_hbm.at[pl.ds(blk_off, _QIDXBLK)], s0)
        pltpu.sync_copy(i1_hbm.at[pl.ds(blk_off, _QIDXBLK)], s1)
        pltpu.sync_copy(i2_hbm.at[pl.ds(blk_off, _QIDXBLK)], s2)
        pltpu.sync_copy(i3_hbm.at[pl.ds(blk_off, _QIDXBLK)], s3)

        def qbuild(g, carry2):
            o = g * 16
            a = s0[pl.ds(o, 16)]
            b_ = s1[pl.ds(o, 16)]
            c_ = s2[pl.ds(o, 16)]
            d_ = s3[pl.ds(o, 16)]
            qidx_v[pl.ds(o, 16)] = a * 125 + b_ * 25 + c_ * 5 + d_
            return carry2

        lax.fori_loop(0, _QIDXBLK // 16, qbuild, 0)

        def pair(p, carry2):
            for b in range(2):
                qloc = pl.multiple_of((p * 2 + b) * _QC, 64)
                qoff = pl.multiple_of(blk_off + qloc, 64)

                @pl.when((ob > 0) | (p > 0))
                def _wait_prev_write():
                    pltpu.make_async_copy(
                        rows[b], out_hbm.at[pl.ds(qoff, _QC)],
                        wsem[b]).wait()

                pltpu.async_copy(
                    qtable_hbm.at[qidx_v.at[pl.ds(qloc, _QC)]],
                    rows[b],
                    gsem[b],
                ).wait()
                pltpu.async_copy(
                    rows[b], out_hbm.at[pl.ds(qoff, _QC)], wsem[b])
            return carry2

        lax.fori_loop(0, _PAIRS, pair, 0)
        return carry

    lax.fori_loop(0, _NBLK, blk, 0)
    for b in range(2):
        pltpu.make_async_copy(
            rows[b], out_hbm.at[pl.ds(pl.multiple_of(qbase, 64), _QC)],
            wsem[b]).wait()


@jax.jit
def _emb(i0, i1, i2, i3, qtable):
    mesh = plsc.VectorSubcoreMesh(core_axis_name="c", subcore_axis_name="s")
    f = pl.kernel(
        _emb_body,
        mesh=mesh,
        out_type=jax.ShapeDtypeStruct((_NQ, 4 * _D), jnp.float32),
        scratch_types=[
            pltpu.VMEM((_QIDXBLK,), jnp.int32),
            pltpu.VMEM((_QIDXBLK,), jnp.int32),
            pltpu.VMEM((_QIDXBLK,), jnp.int32),
            pltpu.VMEM((_QIDXBLK,), jnp.int32),
            pltpu.VMEM((_QIDXBLK,), jnp.int32),
            pltpu.VMEM((_QC, 4 * _D), jnp.float32),
            pltpu.VMEM((_QC, 4 * _D), jnp.float32),
            pltpu.SemaphoreType.DMA,
            pltpu.SemaphoreType.DMA,
            pltpu.SemaphoreType.DMA,
            pltpu.SemaphoreType.DMA,
        ],
    )
    return f(i0, i1, i2, i3, qtable)


def kernel(sequence_int, table):
    idx = sequence_int.reshape(_NQ, 4)
    i0, i1, i2, i3 = idx[:, 0], idx[:, 1], idx[:, 2], idx[:, 3]
    r = jnp.arange(625)
    qtable = jnp.concatenate(
        [table[(r // 125) % 5], table[(r // 25) % 5],
         table[(r // 5) % 5], table[r % 5]], axis=1)
    out = _emb(i0, i1, i2, i3, qtable)
    return out.reshape(_B, _L, _D)


# speedup vs baseline: 5.3545x; 5.3545x over previous
import jax
import jax.numpy as jnp
from jax import lax
from jax.experimental import pallas as pl
from jax.experimental.pallas import tpu as pltpu
from jax.experimental.pallas import tpu_sc as plsc

_B, _L, _D = 1024, 2048, 128
_N = _B * _L
_NC, _NS = 2, 16
_NW = _NC * _NS
_NPW = _N // _NW
_C = 256
_KSUB = _C // 128
_IDXBLK = 8192
_NBLK = _NPW // _IDXBLK
_PAIRS = _IDXBLK // (2 * _C)


def _emb_body(idx_hbm, table_hbm, out_hbm,
              table_sh, idx_v, rows0, rows1, gsem0, gsem1, wsem0, wsem1):
    cid = lax.axis_index("c")
    sid = lax.axis_index("s")
    wid = sid * _NC + cid
    base = wid * _NPW
    rows = (rows0, rows1)
    gsem = (gsem0, gsem1)
    wsem = (wsem0, wsem1)

    @pl.when(sid == 0)
    def _fill_table():
        pltpu.sync_copy(table_hbm, table_sh)

    plsc.subcore_barrier()

    def blk(ob, carry):
        blk_off = base + ob * _IDXBLK
        pltpu.sync_copy(idx_hbm.at[pl.ds(blk_off, _IDXBLK)], idx_v)

        def pair(p, carry2):
            for b in range(2):
                loc = (p * 2 + b) * _C
                off = blk_off + loc

                @pl.when((ob > 0) | (p > 0))
                def _wait_prev_write():
                    pltpu.make_async_copy(
                        rows[b], out_hbm.at[pl.ds(off, _C)], wsem[b]).wait()

                cps = [
                    pltpu.async_copy(
                        table_sh.at[idx_v.at[pl.ds(loc + j * 128, 128)]],
                        rows[b].at[pl.ds(j * 128, 128)],
                        gsem[b],
                    )
                    for j in range(_KSUB)
                ]
                for cp in cps:
                    cp.wait()
                pltpu.async_copy(rows[b], out_hbm.at[pl.ds(off, _C)], wsem[b])
            return carry2

        lax.fori_loop(0, _PAIRS, pair, 0)
        return carry

    lax.fori_loop(0, _NBLK, blk, 0)
    for b in range(2):
        pltpu.make_async_copy(
            rows[b], out_hbm.at[pl.ds(base, _C)], wsem[b]).wait()


@jax.jit
def _emb(idx2, table):
    mesh = plsc.VectorSubcoreMesh(core_axis_name="c", subcore_axis_name="s")
    f = pl.kernel(
        _emb_body,
        mesh=mesh,
        out_type=jax.ShapeDtypeStruct((_N, _D), jnp.float32),
        scratch_types=[
            pltpu.VMEM_SHARED((5, _D), jnp.float32),
            pltpu.VMEM((_IDXBLK,), jnp.int32),
            pltpu.VMEM((_C, _D), jnp.float32),
            pltpu.VMEM((_C, _D), jnp.float32),
            pltpu.SemaphoreType.DMA,
            pltpu.SemaphoreType.DMA,
            pltpu.SemaphoreType.DMA,
            pltpu.SemaphoreType.DMA,
        ],
    )
    return f(idx2, table)


def kernel(sequence_int, table):
    idx2 = sequence_int.reshape(_N)
    out = _emb(idx2, table)
    return out.reshape(_B, _L, _D)


# speedup vs baseline: 5.4033x; 1.0091x over previous
import jax
import jax.numpy as jnp
from jax import lax
from jax.experimental import pallas as pl
from jax.experimental.pallas import tpu as pltpu
from jax.experimental.pallas import tpu_sc as plsc

_B, _L, _D = 1024, 2048, 128
_N = _B * _L
_NC, _NS = 2, 16
_NW = _NC * _NS
_NPW = _N // _NW
_C = 128
_NBUF = 4
_IDXBLK = 8192
_NBLK = _NPW // _IDXBLK
_GRPS = _IDXBLK // (_NBUF * _C)


def _emb_body(idx_hbm, table_hbm, out_hbm,
              table_sh, idx_v, rows0, rows1, rows2, rows3,
              gsem0, gsem1, gsem2, gsem3, wsem0, wsem1, wsem2, wsem3):
    cid = lax.axis_index("c")
    sid = lax.axis_index("s")
    wid = sid * _NC + cid
    base = wid * _NPW
    rows = (rows0, rows1, rows2, rows3)
    gsem = (gsem0, gsem1, gsem2, gsem3)
    wsem = (wsem0, wsem1, wsem2, wsem3)

    @pl.when(sid == 0)
    def _fill_table():
        pltpu.sync_copy(table_hbm, table_sh)

    plsc.subcore_barrier()

    def blk(ob, carry):
        blk_off = base + ob * _IDXBLK
        pltpu.sync_copy(idx_hbm.at[pl.ds(blk_off, _IDXBLK)], idx_v)

        def grp(p, carry2):
            for b in range(_NBUF):
                loc = (p * _NBUF + b) * _C
                off = blk_off + loc

                @pl.when((ob > 0) | (p > 0))
                def _wait_prev_write():
                    pltpu.make_async_copy(
                        rows[b], out_hbm.at[pl.ds(off, _C)], wsem[b]).wait()

                pltpu.async_copy(
                    table_sh.at[idx_v.at[pl.ds(loc, _C)]],
                    rows[b],
                    gsem[b],
                )
            for b in range(_NBUF):
                loc = (p * _NBUF + b) * _C
                off = blk_off + loc
                pltpu.make_async_copy(
                    table_sh.at[idx_v.at[pl.ds(loc, _C)]],
                    rows[b], gsem[b]).wait()
                pltpu.async_copy(rows[b], out_hbm.at[pl.ds(off, _C)], wsem[b])
            return carry2

        lax.fori_loop(0, _GRPS, grp, 0)
        return carry

    lax.fori_loop(0, _NBLK, blk, 0)
    for b in range(_NBUF):
        pltpu.make_async_copy(
            rows[b], out_hbm.at[pl.ds(base, _C)], wsem[b]).wait()


@jax.jit
def _emb(idx2, table):
    mesh = plsc.VectorSubcoreMesh(core_axis_name="c", subcore_axis_name="s")
    f = pl.kernel(
        _emb_body,
        mesh=mesh,
        out_type=jax.ShapeDtypeStruct((_N, _D), jnp.float32),
        scratch_types=[
            pltpu.VMEM_SHARED((5, _D), jnp.float32),
            pltpu.VMEM((_IDXBLK,), jnp.int32),
            pltpu.VMEM((_C, _D), jnp.float32),
            pltpu.VMEM((_C, _D), jnp.float32),
            pltpu.VMEM((_C, _D), jnp.float32),
            pltpu.VMEM((_C, _D), jnp.float32),
            pltpu.SemaphoreType.DMA,
            pltpu.SemaphoreType.DMA,
            pltpu.SemaphoreType.DMA,
            pltpu.SemaphoreType.DMA,
            pltpu.SemaphoreType.DMA,
            pltpu.SemaphoreType.DMA,
            pltpu.SemaphoreType.DMA,
            pltpu.SemaphoreType.DMA,
        ],
    )
    return f(idx2, table)


def kernel(sequence_int, table):
    idx2 = sequence_int.reshape(_N)
    out = _emb(idx2, table)
    return out.reshape(_B, _L, _D)


# speedup vs baseline: 6.1573x; 1.1396x over previous
import jax
import jax.numpy as jnp
from jax import lax
from jax.experimental import pallas as pl
from jax.experimental.pallas import tpu as pltpu
from jax.experimental.pallas import tpu_sc as plsc

_B, _L, _D = 1024, 2048, 128
_N = _B * _L
_NC, _NS = 2, 16
_NW = _NC * _NS
_NPW = _N // _NW
_C = 128
_NBUF = 4
_IDXBLK = 8192
_NBLK = _NPW // _IDXBLK
_GRPS = _IDXBLK // (_NBUF * _C)


def _emb_body(idx_hbm, table_hbm, out_hbm,
              table_sh, idx_v, rows0, rows1, rows2, rows3,
              gsem0, gsem1, gsem2, gsem3, wsem0, wsem1, wsem2, wsem3):
    cid = lax.axis_index("c")
    sid = lax.axis_index("s")
    wid = sid * _NC + cid
    base = wid * _NPW
    rows = (rows0, rows1, rows2, rows3)
    gsem = (gsem0, gsem1, gsem2, gsem3)
    wsem = (wsem0, wsem1, wsem2, wsem3)

    @pl.when(sid == 0)
    def _fill_table():
        pltpu.sync_copy(table_hbm, table_sh)

    plsc.subcore_barrier()
    woff = sid * 5

    def blk(ob, carry):
        blk_off = base + ob * _IDXBLK
        pltpu.sync_copy(idx_hbm.at[pl.ds(blk_off, _IDXBLK)], idx_v)
        for i in range(_IDXBLK // 16):
            idx_v[pl.ds(i * 16, 16)] = idx_v[pl.ds(i * 16, 16)] + woff

        def grp(p, carry2):
            for b in range(_NBUF):
                loc = (p * _NBUF + b) * _C
                off = blk_off + loc

                @pl.when((ob > 0) | (p > 0))
                def _wait_prev_write():
                    pltpu.make_async_copy(
                        rows[b], out_hbm.at[pl.ds(off, _C)], wsem[b]).wait()

                pltpu.async_copy(
                    table_sh.at[idx_v.at[pl.ds(loc, _C)]],
                    rows[b],
                    gsem[b],
                )
            for b in range(_NBUF):
                loc = (p * _NBUF + b) * _C
                off = blk_off + loc
                pltpu.make_async_copy(
                    table_sh.at[idx_v.at[pl.ds(loc, _C)]],
                    rows[b], gsem[b]).wait()
                pltpu.async_copy(rows[b], out_hbm.at[pl.ds(off, _C)], wsem[b])
            return carry2

        lax.fori_loop(0, _GRPS, grp, 0)
        return carry

    lax.fori_loop(0, _NBLK, blk, 0)
    for b in range(_NBUF):
        pltpu.make_async_copy(
            rows[b], out_hbm.at[pl.ds(base, _C)], wsem[b]).wait()


@jax.jit
def _emb(idx2, table):
    mesh = plsc.VectorSubcoreMesh(core_axis_name="c", subcore_axis_name="s")
    f = pl.kernel(
        _emb_body,
        mesh=mesh,
        out_type=jax.ShapeDtypeStruct((_N, _D), jnp.float32),
        scratch_types=[
            pltpu.VMEM_SHARED((5 * _NS, _D), jnp.float32),
            pltpu.VMEM((_IDXBLK,), jnp.int32),
            pltpu.VMEM((_C, _D), jnp.float32),
            pltpu.VMEM((_C, _D), jnp.float32),
            pltpu.VMEM((_C, _D), jnp.float32),
            pltpu.VMEM((_C, _D), jnp.float32),
            pltpu.SemaphoreType.DMA,
            pltpu.SemaphoreType.DMA,
            pltpu.SemaphoreType.DMA,
            pltpu.SemaphoreType.DMA,
            pltpu.SemaphoreType.DMA,
            pltpu.SemaphoreType.DMA,
            pltpu.SemaphoreType.DMA,
            pltpu.SemaphoreType.DMA,
        ],
    )
    return f(idx2, table)


def kernel(sequence_int, table):
    idx2 = sequence_int.reshape(_N)
    table_rep = jnp.tile(table, (_NS, 1))
    out = _emb(idx2, table_rep)
    return out.reshape(_B, _L, _D)
